# Initial kernel scaffold; baseline (speedup 1.0000x reference)
import jax
import jax.numpy as jnp
from jax.experimental import pallas as pl


def kernel(x, symbols):
    raise NotImplementedError("write your pallas kernel here")


# speedup vs baseline: 2708.0353x; 2708.0353x over previous
import functools

import numpy as np
import jax
import jax.numpy as jnp
from jax import lax
from jax.experimental import pallas as pl
from jax.experimental.pallas import tpu as pltpu
from jax.experimental.pallas import tpu_sc as plsc

A = 64
V = 32
L = 2048
B = 4096

NW = 32
RPT = B // NW
CH = 16
NCH = RPT // CH


def _build_perms():
    pk = jax.random.key(7)
    keys = jax.random.split(pk, B)
    return jax.vmap(lambda k: jax.random.permutation(k, A))(keys)


_PERMS_NP = None


def _perms_flat_operand():
    global _PERMS_NP
    if _PERMS_NP is None:
        try:
            with jax.default_device(jax.devices("cpu")[0]):
                p = jax.jit(_build_perms)()
                _PERMS_NP = np.asarray(jax.device_get(p)).astype(np.int32).reshape(-1)
        except Exception:
            return _build_perms().reshape(-1).astype(jnp.int32)
    return jnp.asarray(_PERMS_NP)


def _argmax_body(x_ref, sel_ref, o_ref):
    xb = x_ref[...]
    r = xb.shape[0]
    lane = lax.broadcasted_iota(jnp.int32, (r, L), 1)
    idx = (lane & (V - 1)).astype(jnp.float32)
    v = xb
    for s in (1, 2, 4, 8, 16):
        vr = pltpu.roll(v, L - s, axis=1)
        ir = pltpu.roll(idx, L - s, axis=1)
        take = v >= vr
        v = jnp.where(take, v, vr)
        idx = jnp.where(take, idx, ir)
    vals = lax.dot_general(idx, sel_ref[...],
                           (((1,), (0,)), ((), ())),
                           preferred_element_type=jnp.float32)
    o_ref[...] = vals.astype(jnp.int32)


def _grouped_argmax(x):
    R = 256
    sel = np.zeros((L, A), np.float32)
    sel[np.arange(A) * V, np.arange(A)] = 1.0
    return pl.pallas_call(
        _argmax_body,
        grid=(B // R,),
        in_specs=[
            pl.BlockSpec((R, L), lambda i: (i, 0)),
            pl.BlockSpec((L, A), lambda i: (0, 0)),
        ],
        out_specs=pl.BlockSpec((R, A), lambda i: (i, 0)),
        out_shape=jax.ShapeDtypeStruct((B, A), jnp.int32),
    )(x, jnp.asarray(sel))


def _fill_body(vals_hbm, perms_hbm, syms_hbm, out_hbm, vals_v, perms_v,
               syms_v, buf, sem):
    wid = lax.axis_index("c") * 16 + lax.axis_index("s")
    base = wid * RPT
    pltpu.sync_copy(vals_hbm.at[pl.ds(base * A, RPT * A)], vals_v)
    pltpu.sync_copy(perms_hbm.at[pl.ds(base * A, RPT * A)], perms_v)
    pltpu.sync_copy(syms_hbm, syms_v)

    zero16 = jnp.zeros((16,), jnp.int32)

    @pl.loop(0, NCH)
    def _chunk(c):
        for i in range(CH * L // 16):
            buf[pl.ds(i * 16, 16)] = zero16
        for r in range(CH):
            lr = c * CH + r
            rb = r * L
            carry = jnp.int32(0)
            for k in range(A // 16):
                pv = perms_v[pl.ds(lr * A + k * 16, 16)]
                lens = plsc.load_gather(vals_v, [pv + lr * A])
                syms = plsc.load_gather(syms_v, [pv])
                ends = plsc.cumsum(lens) + carry
                carry = carry + jnp.sum(lens)
                starts = ends - lens
                for o in range(V - 1):
                    plsc.store_scatter(buf, [starts + (rb + o)], syms,
                                       mask=lens > o)
        pltpu.sync_copy(buf, out_hbm.at[pl.ds((base + c * CH) * L, CH * L)])


def _ragged_fill(vals_flat, perms_flat, symbols):
    kern = functools.partial(
        pl.kernel,
        out_type=jax.ShapeDtypeStruct((B * L,), jnp.int32),
        mesh=plsc.VectorSubcoreMesh(core_axis_name="c", subcore_axis_name="s"),
        scratch_types=[
            pltpu.VMEM((RPT * A,), jnp.int32),
            pltpu.VMEM((RPT * A,), jnp.int32),
            pltpu.VMEM((A,), jnp.int32),
            pltpu.VMEM((CH * L,), jnp.int32),
            pltpu.SemaphoreType.DMA,
        ],
        compiler_params=pltpu.CompilerParams(needs_layout_passes=False),
    )(_fill_body)
    return kern(vals_flat, perms_flat, symbols)


def kernel(x, symbols):
    vals = _grouped_argmax(x)
    perms_flat = _perms_flat_operand()
    out_flat = _ragged_fill(vals.reshape(-1), perms_flat, symbols)
    result = out_flat.reshape(B, L)
    zeros = jnp.zeros((B, L), jnp.float32)
    return (result, zeros, zeros)
